# Initial kernel scaffold; baseline (speedup 1.0000x reference)
import jax
import jax.numpy as jnp
from jax.experimental import pallas as pl


def kernel(x, edge_index, batch, W0, b0, W1, b1, W2, b2, W3, b3, Wl, bl):
    raise NotImplementedError("write your pallas kernel here")


# speedup vs baseline: 3.6925x; 3.6925x over previous
import jax
import jax.numpy as jnp
from jax import lax
from jax.experimental import pallas as pl
from jax.experimental.pallas import tpu as pltpu
from jax.experimental.pallas import tpu_sc as plsc

N = 10000
E = 320000
F_IN = 128
H = 256
G = 64
C = 10
ALPHA = 0.01

NC = 2
NS = 16
NT = NC * NS
SLAB = 320
LAST = N - (NT - 1) * SLAB
DUMMY = SLAB
SHIFT = 512
CHE = 2048
NSCAN = (E + CHE - 1) // CHE
REME = E - (NSCAN - 1) * CHE
GSUB = 128
PCAP = CHE + 2 * GSUB
CAP = E + GSUB

_HIGH = lax.Precision.HIGHEST
_MESH = plsc.VectorSubcoreMesh(core_axis_name="c", subcore_axis_name="s")


def _prep_body(src_hbm, dst_hbm, lists_hbm, counts_hbm, deg_hbm,
               srcbuf, dstbuf, pend, lbuf, cntbuf, acc16):
    c = lax.axis_index("c")
    s = lax.axis_index("s")
    t = c * NS + s
    lo = t * SLAB
    listbase = t * CAP

    def scan_chunk(ci, carry):
        pcnt, fl = carry
        off = pl.multiple_of(ci * CHE, CHE)

        @pl.when(ci < NSCAN - 1)
        def _():
            pltpu.sync_copy(src_hbm.at[pl.ds(off, CHE)], srcbuf.at[pl.ds(0, CHE)])
            pltpu.sync_copy(dst_hbm.at[pl.ds(off, CHE)], dstbuf.at[pl.ds(0, CHE)])

        @pl.when(ci == NSCAN - 1)
        def _():
            pltpu.sync_copy(src_hbm.at[pl.ds(off, REME)], srcbuf.at[pl.ds(0, REME)])
            pltpu.sync_copy(dst_hbm.at[pl.ds(off, REME)], dstbuf.at[pl.ds(0, REME)])

        ngroups = jnp.where(ci < NSCAN - 1, CHE // 16, REME // 16)

        def group(j, pcnt2):
            d = dstbuf[pl.ds(16 * j, 16)]
            sv = srcbuf[pl.ds(16 * j, 16)]
            u = d - lo
            oob = lax.shift_right_logical(u | (SLAB - 1 - u), 31)
            comb = sv * SHIFT + u
            for jl in range(16):
                pend[pl.ds(pcnt2, 16)] = jnp.broadcast_to(comb[jl], (16,))
                pcnt2 = pcnt2 + (1 - oob[jl])
            return pcnt2

        pcnt = lax.fori_loop(0, ngroups, group, pcnt)
        nblk = pcnt // GSUB

        def fb(b, fl2):
            pltpu.sync_copy(pend.at[pl.ds(b * GSUB, GSUB)],
                            lists_hbm.at[pl.ds(listbase + fl2 * GSUB, GSUB)])
            return fl2 + 1

        fl = lax.fori_loop(0, nblk, fb, fl)
        rbase = nblk * GSUB
        for g in range(GSUB // 16):
            pend[pl.ds(16 * g, 16)] = pend[pl.ds(rbase + 16 * g, 16)]
        return pcnt - rbase, fl

    pcnt, fl = lax.fori_loop(0, NSCAN, scan_chunk,
                             (jnp.int32(0), jnp.int32(0)))

    dummyv = jnp.full((16,), DUMMY, jnp.int32)
    for g in range(GSUB // 16):
        pend[pl.ds(pcnt + 16 * g, 16)] = dummyv

    @pl.when(pcnt > 0)
    def _():
        pltpu.sync_copy(pend.at[pl.ds(0, GSUB)],
                        lists_hbm.at[pl.ds(listbase + fl * GSUB, GSUB)])

    flf = jnp.where(pcnt > 0, fl + 1, fl)
    cntbuf[pl.ds(0, 16)] = jnp.broadcast_to(flf * GSUB, (16,))
    pltpu.sync_copy(cntbuf.at[pl.ds(0, 16)], counts_hbm.at[pl.ds(t * 16, 16)])

    ones16 = jnp.ones((16,), jnp.float32)

    def initrow(r, cy):
        acc16[r, :] = ones16
        return cy

    lax.fori_loop(0, SLAB + 1, initrow, 0)

    def degblk(b, cy):
        pltpu.sync_copy(lists_hbm.at[pl.ds(listbase + b * GSUB, GSUB)],
                        lbuf.at[pl.ds(0, GSUB)])

        def deggrp(g, cy2):
            ld16 = lbuf[pl.ds(16 * g, 16)] & (SHIFT - 1)
            for jl in range(16):
                ld = ld16[jl]
                acc16[ld, :] = acc16[ld, :] + ones16
            return cy2

        return lax.fori_loop(0, GSUB // 16, deggrp, cy)

    lax.fori_loop(0, flf, degblk, 0)

    @pl.when(t < NT - 1)
    def _():
        pltpu.sync_copy(acc16.at[pl.ds(0, SLAB)], deg_hbm.at[pl.ds(lo, SLAB)])

    @pl.when(t == NT - 1)
    def _():
        pltpu.sync_copy(acc16.at[pl.ds(0, LAST)], deg_hbm.at[pl.ds(lo, LAST)])


_sc_prep = pl.kernel(
    _prep_body,
    out_type=(
        jax.ShapeDtypeStruct((NT * CAP,), jnp.int32),
        jax.ShapeDtypeStruct((NT * 16,), jnp.int32),
        jax.ShapeDtypeStruct((N, 16), jnp.float32),
    ),
    mesh=_MESH,
    scratch_types=[
        pltpu.VMEM((CHE,), jnp.int32),
        pltpu.VMEM((CHE,), jnp.int32),
        pltpu.VMEM((PCAP,), jnp.int32),
        pltpu.VMEM((GSUB,), jnp.int32),
        pltpu.VMEM((16,), jnp.int32),
        pltpu.VMEM((SLAB + 1, 16), jnp.float32),
    ],
)


def _agg_body(hp_hbm, lists_hbm, counts_hbm, out_hbm,
              lbuf, srcbuf, cntbuf, rows, acc, sem):
    c = lax.axis_index("c")
    s = lax.axis_index("s")
    t = c * NS + s
    lo = t * SLAB
    listbase = t * CAP

    @pl.when(t < NT - 1)
    def _():
        pltpu.sync_copy(hp_hbm.at[pl.ds(lo, SLAB)], acc.at[pl.ds(0, SLAB)])

    @pl.when(t == NT - 1)
    def _():
        pltpu.sync_copy(hp_hbm.at[pl.ds(lo, LAST)], acc.at[pl.ds(0, LAST)])

    pltpu.sync_copy(counts_hbm.at[pl.ds(t * 16, 16)], cntbuf.at[pl.ds(0, 16)])
    nblk = cntbuf[pl.ds(0, 16)][0] // GSUB

    def blk(b, cy):
        pltpu.sync_copy(lists_hbm.at[pl.ds(listbase + b * GSUB, GSUB)],
                        lbuf.at[pl.ds(0, GSUB)])
        for g in range(GSUB // 16):
            srcbuf[pl.ds(16 * g, 16)] = jnp.right_shift(lbuf[pl.ds(16 * g, 16)], 9)
        pltpu.async_copy(hp_hbm.at[srcbuf], rows, sem).wait()

        def grp(g, cy2):
            ld16 = lbuf[pl.ds(16 * g, 16)] & (SHIFT - 1)
            for jl in range(16):
                ld = ld16[jl]
                for f in range(H // 16):
                    sl = pl.ds(16 * f, 16)
                    acc[ld, sl] = acc[ld, sl] + rows[16 * g + jl, sl]
            return cy2

        return lax.fori_loop(0, GSUB // 16, grp, cy)

    lax.fori_loop(0, nblk, blk, 0)

    @pl.when(t < NT - 1)
    def _():
        pltpu.sync_copy(acc.at[pl.ds(0, SLAB)], out_hbm.at[pl.ds(lo, SLAB)])

    @pl.when(t == NT - 1)
    def _():
        pltpu.sync_copy(acc.at[pl.ds(0, LAST)], out_hbm.at[pl.ds(lo, LAST)])


_sc_agg = pl.kernel(
    _agg_body,
    out_type=jax.ShapeDtypeStruct((N, H), jnp.float32),
    mesh=_MESH,
    scratch_types=[
        pltpu.VMEM((GSUB,), jnp.int32),
        pltpu.VMEM((GSUB,), jnp.int32),
        pltpu.VMEM((16,), jnp.int32),
        pltpu.VMEM((GSUB, H), jnp.float32),
        pltpu.VMEM((SLAB + 1, H), jnp.float32),
        pltpu.SemaphoreType.DMA,
    ],
)

BN = 1000
GRID = N // BN


def _first_body(x_ref, w_ref, deg_ref, hp_ref, dinv_ref):
    dinv = lax.rsqrt(deg_ref[...])
    y = jnp.dot(x_ref[...], w_ref[...], preferred_element_type=jnp.float32,
                precision=_HIGH)
    hp_ref[...] = y * dinv
    dinv_ref[...] = dinv


_tc_first = pl.pallas_call(
    _first_body,
    grid=(GRID,),
    in_specs=[
        pl.BlockSpec((BN, F_IN), lambda i: (i, 0)),
        pl.BlockSpec((F_IN, H), lambda i: (0, 0)),
        pl.BlockSpec((BN, 1), lambda i: (i, 0)),
    ],
    out_specs=[
        pl.BlockSpec((BN, H), lambda i: (i, 0)),
        pl.BlockSpec((BN, 1), lambda i: (i, 0)),
    ],
    out_shape=[
        jax.ShapeDtypeStruct((N, H), jnp.float32),
        jax.ShapeDtypeStruct((N, 1), jnp.float32),
    ],
)


def _layer_body(s_ref, dinv_ref, b_ref, w_ref, o_ref):
    dinv = dinv_ref[...]
    v = dinv * s_ref[...] + b_ref[...]
    z = jnp.where(v >= 0, v, ALPHA * v)
    y = jnp.dot(z, w_ref[...], preferred_element_type=jnp.float32, precision=_HIGH)
    o_ref[...] = y * dinv


_tc_layer = pl.pallas_call(
    _layer_body,
    grid=(GRID,),
    in_specs=[
        pl.BlockSpec((BN, H), lambda i: (i, 0)),
        pl.BlockSpec((BN, 1), lambda i: (i, 0)),
        pl.BlockSpec((1, H), lambda i: (0, 0)),
        pl.BlockSpec((H, H), lambda i: (0, 0)),
    ],
    out_specs=pl.BlockSpec((BN, H), lambda i: (i, 0)),
    out_shape=jax.ShapeDtypeStruct((N, H), jnp.float32),
)


def _final_body(s_ref, dinv_ref, b_ref, bid_ref, wl1_ref, wl2_ref, bl_ref,
                o_ref, maxa, suma, cnta):
    i = pl.program_id(0)

    @pl.when(i == 0)
    def _():
        maxa[...] = jnp.full((G, H), -jnp.inf, jnp.float32)
        suma[...] = jnp.zeros((G, H), jnp.float32)
        cnta[...] = jnp.zeros((G, 1), jnp.float32)

    v = dinv_ref[...] * s_ref[...] + b_ref[...]
    h = jnp.where(v >= 0, v, ALPHA * v)
    bid = bid_ref[...]
    gids = lax.broadcasted_iota(jnp.int32, (BN, G), 1)
    mask = (bid == gids).astype(jnp.float32)
    suma[...] += lax.dot_general(mask, h, (((0,), (0,)), ((), ())),
                                 preferred_element_type=jnp.float32,
                                 precision=_HIGH)
    onesc = jnp.ones((BN, 1), jnp.float32)
    cnta[...] += lax.dot_general(mask, onesc, (((0,), (0,)), ((), ())),
                                 preferred_element_type=jnp.float32,
                                 precision=_HIGH)
    for g in range(G):
        m = jnp.max(jnp.where(bid == g, h, -jnp.inf), axis=0, keepdims=True)
        maxa[pl.ds(g, 1), :] = jnp.maximum(maxa[pl.ds(g, 1), :], m)

    @pl.when(i == GRID - 1)
    def _():
        maxp = maxa[...]
        maxp = jnp.where(jnp.isfinite(maxp), maxp, 0.0)
        meanp = suma[...] / jnp.maximum(cnta[...], 1.0)
        o_ref[...] = (
            jnp.dot(maxp, wl1_ref[...], preferred_element_type=jnp.float32,
                    precision=_HIGH)
            + jnp.dot(meanp, wl2_ref[...], preferred_element_type=jnp.float32,
                      precision=_HIGH)
            + bl_ref[...]
        )


_tc_final = pl.pallas_call(
    _final_body,
    grid=(GRID,),
    in_specs=[
        pl.BlockSpec((BN, H), lambda i: (i, 0)),
        pl.BlockSpec((BN, 1), lambda i: (i, 0)),
        pl.BlockSpec((1, H), lambda i: (0, 0)),
        pl.BlockSpec((BN, 1), lambda i: (i, 0)),
        pl.BlockSpec((H, C), lambda i: (0, 0)),
        pl.BlockSpec((H, C), lambda i: (0, 0)),
        pl.BlockSpec((1, C), lambda i: (0, 0)),
    ],
    out_specs=pl.BlockSpec((G, C), lambda i: (0, 0)),
    out_shape=jax.ShapeDtypeStruct((G, C), jnp.float32),
    scratch_shapes=[
        pltpu.VMEM((G, H), jnp.float32),
        pltpu.VMEM((G, H), jnp.float32),
        pltpu.VMEM((G, 1), jnp.float32),
    ],
)


def kernel(x, edge_index, batch, W0, b0, W1, b1, W2, b2, W3, b3, Wl, bl):
    src = edge_index[0]
    dst = edge_index[1]
    lists, counts, deg16 = _sc_prep(src, dst)
    deg = deg16[:, 0:1]
    hp, dinv = _tc_first(x, W0, deg)
    s = _sc_agg(hp, lists, counts)
    hp = _tc_layer(s, dinv, b0.reshape(1, H), W1)
    s = _sc_agg(hp, lists, counts)
    hp = _tc_layer(s, dinv, b1.reshape(1, H), W2)
    s = _sc_agg(hp, lists, counts)
    hp = _tc_layer(s, dinv, b2.reshape(1, H), W3)
    s = _sc_agg(hp, lists, counts)
    return _tc_final(s, dinv, b3.reshape(1, H), batch.reshape(N, 1),
                     Wl[:H], Wl[H:], bl.reshape(1, C))


# speedup vs baseline: 4.0785x; 1.1045x over previous
import jax
import jax.numpy as jnp
from jax import lax
from jax.experimental import pallas as pl
from jax.experimental.pallas import tpu as pltpu
from jax.experimental.pallas import tpu_sc as plsc

N = 10000
E = 320000
F_IN = 128
H = 256
G = 64
C = 10
ALPHA = 0.01

NC = 2
NS = 16
NT = NC * NS
SLAB = 320
LAST = N - (NT - 1) * SLAB
DUMMY = SLAB
SHIFT = 512
CHE = 2048
NSCAN = (E + CHE - 1) // CHE
REME = E - (NSCAN - 1) * CHE
GSUB = 64
PCAP = CHE + 2 * GSUB
CAP = E + GSUB

_HIGH = lax.Precision.HIGHEST
_MESH = plsc.VectorSubcoreMesh(core_axis_name="c", subcore_axis_name="s")


def _prep_body(src_hbm, dst_hbm, lists_hbm, counts_hbm, deg_hbm,
               srcbuf, dstbuf, pend, lbuf, cntbuf, acc16):
    c = lax.axis_index("c")
    s = lax.axis_index("s")
    t = c * NS + s
    lo = t * SLAB
    listbase = t * CAP

    def scan_chunk(ci, carry):
        pcnt, fl = carry
        off = pl.multiple_of(ci * CHE, CHE)

        @pl.when(ci < NSCAN - 1)
        def _():
            pltpu.sync_copy(src_hbm.at[pl.ds(off, CHE)], srcbuf.at[pl.ds(0, CHE)])
            pltpu.sync_copy(dst_hbm.at[pl.ds(off, CHE)], dstbuf.at[pl.ds(0, CHE)])

        @pl.when(ci == NSCAN - 1)
        def _():
            pltpu.sync_copy(src_hbm.at[pl.ds(off, REME)], srcbuf.at[pl.ds(0, REME)])
            pltpu.sync_copy(dst_hbm.at[pl.ds(off, REME)], dstbuf.at[pl.ds(0, REME)])

        ngroups = jnp.where(ci < NSCAN - 1, CHE // 16, REME // 16)

        def group(j, pcnt2):
            d = dstbuf[pl.ds(16 * j, 16)]
            sv = srcbuf[pl.ds(16 * j, 16)]
            u = d - lo
            oob = lax.shift_right_logical(u | (SLAB - 1 - u), 31)
            comb = sv * SHIFT + u
            for jl in range(16):
                pend[pl.ds(pcnt2, 16)] = jnp.broadcast_to(comb[jl], (16,))
                pcnt2 = pcnt2 + (1 - oob[jl])
            return pcnt2

        pcnt = lax.fori_loop(0, ngroups, group, pcnt)
        nblk = pcnt // GSUB

        def fb(b, fl2):
            pltpu.sync_copy(pend.at[pl.ds(b * GSUB, GSUB)],
                            lists_hbm.at[pl.ds(listbase + fl2 * GSUB, GSUB)])
            return fl2 + 1

        fl = lax.fori_loop(0, nblk, fb, fl)
        rbase = nblk * GSUB
        for g in range(GSUB // 16):
            pend[pl.ds(16 * g, 16)] = pend[pl.ds(rbase + 16 * g, 16)]
        return pcnt - rbase, fl

    pcnt, fl = lax.fori_loop(0, NSCAN, scan_chunk,
                             (jnp.int32(0), jnp.int32(0)))

    dummyv = jnp.full((16,), DUMMY, jnp.int32)
    for g in range(GSUB // 16):
        pend[pl.ds(pcnt + 16 * g, 16)] = dummyv

    @pl.when(pcnt > 0)
    def _():
        pltpu.sync_copy(pend.at[pl.ds(0, GSUB)],
                        lists_hbm.at[pl.ds(listbase + fl * GSUB, GSUB)])

    flf = jnp.where(pcnt > 0, fl + 1, fl)
    cntbuf[pl.ds(0, 16)] = jnp.broadcast_to(flf * GSUB, (16,))
    pltpu.sync_copy(cntbuf.at[pl.ds(0, 16)], counts_hbm.at[pl.ds(t * 16, 16)])

    ones16 = jnp.ones((16,), jnp.float32)

    def initrow(r, cy):
        acc16[r, :] = ones16
        return cy

    lax.fori_loop(0, SLAB + 1, initrow, 0)

    def degblk(b, cy):
        pltpu.sync_copy(lists_hbm.at[pl.ds(listbase + b * GSUB, GSUB)],
                        lbuf.at[pl.ds(0, GSUB)])

        def deggrp(g, cy2):
            ld16 = lbuf[pl.ds(16 * g, 16)] & (SHIFT - 1)
            for jl in range(16):
                ld = ld16[jl]
                acc16[ld, :] = acc16[ld, :] + ones16
            return cy2

        return lax.fori_loop(0, GSUB // 16, deggrp, cy)

    lax.fori_loop(0, flf, degblk, 0)

    @pl.when(t < NT - 1)
    def _():
        pltpu.sync_copy(acc16.at[pl.ds(0, SLAB)], deg_hbm.at[pl.ds(lo, SLAB)])

    @pl.when(t == NT - 1)
    def _():
        pltpu.sync_copy(acc16.at[pl.ds(0, LAST)], deg_hbm.at[pl.ds(lo, LAST)])


_sc_prep = pl.kernel(
    _prep_body,
    out_type=(
        jax.ShapeDtypeStruct((NT * CAP,), jnp.int32),
        jax.ShapeDtypeStruct((NT * 16,), jnp.int32),
        jax.ShapeDtypeStruct((N, 16), jnp.float32),
    ),
    mesh=_MESH,
    scratch_types=[
        pltpu.VMEM((CHE,), jnp.int32),
        pltpu.VMEM((CHE,), jnp.int32),
        pltpu.VMEM((PCAP,), jnp.int32),
        pltpu.VMEM((GSUB,), jnp.int32),
        pltpu.VMEM((16,), jnp.int32),
        pltpu.VMEM((SLAB + 1, 16), jnp.float32),
    ],
)


def _agg_body(hp_hbm, lists_hbm, counts_hbm, out_hbm,
              lbufa, lbufb, srca, srcb, cntbuf, rowsa, rowsb, acc, sema, semb):
    c = lax.axis_index("c")
    s = lax.axis_index("s")
    t = c * NS + s
    lo = t * SLAB
    listbase = t * CAP

    @pl.when(t < NT - 1)
    def _():
        pltpu.sync_copy(hp_hbm.at[pl.ds(lo, SLAB)], acc.at[pl.ds(0, SLAB)])

    @pl.when(t == NT - 1)
    def _():
        pltpu.sync_copy(hp_hbm.at[pl.ds(lo, LAST)], acc.at[pl.ds(0, LAST)])

    pltpu.sync_copy(counts_hbm.at[pl.ds(t * 16, 16)], cntbuf.at[pl.ds(0, 16)])
    nblk = cntbuf[pl.ds(0, 16)][0] // GSUB

    def issue(b, lbuf, srcv, rows, sem):
        pltpu.sync_copy(lists_hbm.at[pl.ds(listbase + b * GSUB, GSUB)],
                        lbuf.at[pl.ds(0, GSUB)])
        for g in range(GSUB // 16):
            srcv[pl.ds(16 * g, 16)] = jnp.right_shift(lbuf[pl.ds(16 * g, 16)], 9)
        pltpu.async_copy(hp_hbm.at[srcv], rows, sem)

    def accum(lbuf, rows):
        def grp(g, cy2):
            ld16 = lbuf[pl.ds(16 * g, 16)] & (SHIFT - 1)
            for jl in range(16):
                ld = ld16[jl]
                for f in range(H // 16):
                    sl = pl.ds(16 * f, 16)
                    acc[ld, sl] = acc[ld, sl] + rows[16 * g + jl, sl]
            return cy2

        lax.fori_loop(0, GSUB // 16, grp, 0)

    @pl.when(nblk > 0)
    def _():
        issue(0, lbufa, srca, rowsa, sema)

    def blk(b, cy):
        p = jnp.bitwise_and(b, 1)

        @pl.when((b + 1 < nblk) & (p == 0))
        def _():
            issue(b + 1, lbufb, srcb, rowsb, semb)

        @pl.when((b + 1 < nblk) & (p == 1))
        def _():
            issue(b + 1, lbufa, srca, rowsa, sema)

        @pl.when(p == 0)
        def _():
            pltpu.make_async_copy(hp_hbm.at[srca], rowsa, sema).wait()
            accum(lbufa, rowsa)

        @pl.when(p == 1)
        def _():
            pltpu.make_async_copy(hp_hbm.at[srcb], rowsb, semb).wait()
            accum(lbufb, rowsb)

        return cy

    lax.fori_loop(0, nblk, blk, 0)

    @pl.when(t < NT - 1)
    def _():
        pltpu.sync_copy(acc.at[pl.ds(0, SLAB)], out_hbm.at[pl.ds(lo, SLAB)])

    @pl.when(t == NT - 1)
    def _():
        pltpu.sync_copy(acc.at[pl.ds(0, LAST)], out_hbm.at[pl.ds(lo, LAST)])


_sc_agg = pl.kernel(
    _agg_body,
    out_type=jax.ShapeDtypeStruct((N, H), jnp.float32),
    mesh=_MESH,
    scratch_types=[
        pltpu.VMEM((GSUB,), jnp.int32),
        pltpu.VMEM((GSUB,), jnp.int32),
        pltpu.VMEM((GSUB,), jnp.int32),
        pltpu.VMEM((GSUB,), jnp.int32),
        pltpu.VMEM((16,), jnp.int32),
        pltpu.VMEM((GSUB, H), jnp.float32),
        pltpu.VMEM((GSUB, H), jnp.float32),
        pltpu.VMEM((SLAB + 1, H), jnp.float32),
        pltpu.SemaphoreType.DMA,
        pltpu.SemaphoreType.DMA,
    ],
)

BN = 1000
GRID = N // BN


def _first_body(x_ref, w_ref, deg_ref, hp_ref, dinv_ref):
    dinv = lax.rsqrt(deg_ref[...])
    y = jnp.dot(x_ref[...], w_ref[...], preferred_element_type=jnp.float32,
                precision=_HIGH)
    hp_ref[...] = y * dinv
    dinv_ref[...] = dinv


_tc_first = pl.pallas_call(
    _first_body,
    grid=(GRID,),
    in_specs=[
        pl.BlockSpec((BN, F_IN), lambda i: (i, 0)),
        pl.BlockSpec((F_IN, H), lambda i: (0, 0)),
        pl.BlockSpec((BN, 1), lambda i: (i, 0)),
    ],
    out_specs=[
        pl.BlockSpec((BN, H), lambda i: (i, 0)),
        pl.BlockSpec((BN, 1), lambda i: (i, 0)),
    ],
    out_shape=[
        jax.ShapeDtypeStruct((N, H), jnp.float32),
        jax.ShapeDtypeStruct((N, 1), jnp.float32),
    ],
)


def _layer_body(s_ref, dinv_ref, b_ref, w_ref, o_ref):
    dinv = dinv_ref[...]
    v = dinv * s_ref[...] + b_ref[...]
    z = jnp.where(v >= 0, v, ALPHA * v)
    y = jnp.dot(z, w_ref[...], preferred_element_type=jnp.float32, precision=_HIGH)
    o_ref[...] = y * dinv


_tc_layer = pl.pallas_call(
    _layer_body,
    grid=(GRID,),
    in_specs=[
        pl.BlockSpec((BN, H), lambda i: (i, 0)),
        pl.BlockSpec((BN, 1), lambda i: (i, 0)),
        pl.BlockSpec((1, H), lambda i: (0, 0)),
        pl.BlockSpec((H, H), lambda i: (0, 0)),
    ],
    out_specs=pl.BlockSpec((BN, H), lambda i: (i, 0)),
    out_shape=jax.ShapeDtypeStruct((N, H), jnp.float32),
)


def _final_body(s_ref, dinv_ref, b_ref, bid_ref, wl1_ref, wl2_ref, bl_ref,
                o_ref, maxa, suma, cnta):
    i = pl.program_id(0)

    @pl.when(i == 0)
    def _():
        maxa[...] = jnp.full((G, H), -jnp.inf, jnp.float32)
        suma[...] = jnp.zeros((G, H), jnp.float32)
        cnta[...] = jnp.zeros((G, 1), jnp.float32)

    v = dinv_ref[...] * s_ref[...] + b_ref[...]
    h = jnp.where(v >= 0, v, ALPHA * v)
    bid = bid_ref[...]
    gids = lax.broadcasted_iota(jnp.int32, (BN, G), 1)
    mask = (bid == gids).astype(jnp.float32)
    suma[...] += lax.dot_general(mask, h, (((0,), (0,)), ((), ())),
                                 preferred_element_type=jnp.float32,
                                 precision=_HIGH)
    onesc = jnp.ones((BN, 1), jnp.float32)
    cnta[...] += lax.dot_general(mask, onesc, (((0,), (0,)), ((), ())),
                                 preferred_element_type=jnp.float32,
                                 precision=_HIGH)
    for g in range(G):
        m = jnp.max(jnp.where(bid == g, h, -jnp.inf), axis=0, keepdims=True)
        maxa[pl.ds(g, 1), :] = jnp.maximum(maxa[pl.ds(g, 1), :], m)

    @pl.when(i == GRID - 1)
    def _():
        maxp = maxa[...]
        maxp = jnp.where(jnp.isfinite(maxp), maxp, 0.0)
        meanp = suma[...] / jnp.maximum(cnta[...], 1.0)
        o_ref[...] = (
            jnp.dot(maxp, wl1_ref[...], preferred_element_type=jnp.float32,
                    precision=_HIGH)
            + jnp.dot(meanp, wl2_ref[...], preferred_element_type=jnp.float32,
                      precision=_HIGH)
            + bl_ref[...]
        )


_tc_final = pl.pallas_call(
    _final_body,
    grid=(GRID,),
    in_specs=[
        pl.BlockSpec((BN, H), lambda i: (i, 0)),
        pl.BlockSpec((BN, 1), lambda i: (i, 0)),
        pl.BlockSpec((1, H), lambda i: (0, 0)),
        pl.BlockSpec((BN, 1), lambda i: (i, 0)),
        pl.BlockSpec((H, C), lambda i: (0, 0)),
        pl.BlockSpec((H, C), lambda i: (0, 0)),
        pl.BlockSpec((1, C), lambda i: (0, 0)),
    ],
    out_specs=pl.BlockSpec((G, C), lambda i: (0, 0)),
    out_shape=jax.ShapeDtypeStruct((G, C), jnp.float32),
    scratch_shapes=[
        pltpu.VMEM((G, H), jnp.float32),
        pltpu.VMEM((G, H), jnp.float32),
        pltpu.VMEM((G, 1), jnp.float32),
    ],
)


def kernel(x, edge_index, batch, W0, b0, W1, b1, W2, b2, W3, b3, Wl, bl):
    src = edge_index[0]
    dst = edge_index[1]
    lists, counts, deg16 = _sc_prep(src, dst)
    deg = deg16[:, 0:1]
    hp, dinv = _tc_first(x, W0, deg)
    s = _sc_agg(hp, lists, counts)
    hp = _tc_layer(s, dinv, b0.reshape(1, H), W1)
    s = _sc_agg(hp, lists, counts)
    hp = _tc_layer(s, dinv, b1.reshape(1, H), W2)
    s = _sc_agg(hp, lists, counts)
    hp = _tc_layer(s, dinv, b2.reshape(1, H), W3)
    s = _sc_agg(hp, lists, counts)
    return _tc_final(s, dinv, b3.reshape(1, H), batch.reshape(N, 1),
                     Wl[:H], Wl[H:], bl.reshape(1, C))


# speedup vs baseline: 4.2824x; 1.0500x over previous
import jax
import jax.numpy as jnp
from jax import lax
from jax.experimental import pallas as pl
from jax.experimental.pallas import tpu as pltpu
from jax.experimental.pallas import tpu_sc as plsc

N = 10000
E = 320000
F_IN = 128
H = 256
G = 64
C = 10
ALPHA = 0.01

NC = 2
NS = 16
NT = NC * NS
SLAB = 320
LAST = N - (NT - 1) * SLAB
DUMMY = SLAB
SHIFT = 512
CHE = 2048
NSCAN = (E + CHE - 1) // CHE
REME = E - (NSCAN - 1) * CHE
GSUB = 64
PCAP = CHE + 2 * GSUB
LCHUNK = 1024
LCB = LCHUNK // GSUB
CAP = ((E + GSUB + LCHUNK - 1) // LCHUNK) * LCHUNK

_HIGH = lax.Precision.HIGHEST
_MESH = plsc.VectorSubcoreMesh(core_axis_name="c", subcore_axis_name="s")


def _prep_body(src_hbm, dst_hbm, lists_hbm, counts_hbm, deg_hbm,
               srcbuf, dstbuf, pend, lbuf, cntbuf, acc16):
    c = lax.axis_index("c")
    s = lax.axis_index("s")
    t = c * NS + s
    lo = t * SLAB
    listbase = t * CAP

    def scan_chunk(ci, carry):
        pcnt, fl = carry
        off = pl.multiple_of(ci * CHE, CHE)

        @pl.when(ci < NSCAN - 1)
        def _():
            pltpu.sync_copy(src_hbm.at[pl.ds(off, CHE)], srcbuf.at[pl.ds(0, CHE)])
            pltpu.sync_copy(dst_hbm.at[pl.ds(off, CHE)], dstbuf.at[pl.ds(0, CHE)])

        @pl.when(ci == NSCAN - 1)
        def _():
            pltpu.sync_copy(src_hbm.at[pl.ds(off, REME)], srcbuf.at[pl.ds(0, REME)])
            pltpu.sync_copy(dst_hbm.at[pl.ds(off, REME)], dstbuf.at[pl.ds(0, REME)])

        ngroups = jnp.where(ci < NSCAN - 1, CHE // 16, REME // 16)

        def group(j, pcnt2):
            d = dstbuf[pl.ds(16 * j, 16)]
            sv = srcbuf[pl.ds(16 * j, 16)]
            u = d - lo
            oob = lax.shift_right_logical(u | (SLAB - 1 - u), 31)
            comb = sv * SHIFT + u
            for jl in range(16):
                pend[pl.ds(pcnt2, 16)] = jnp.broadcast_to(comb[jl], (16,))
                pcnt2 = pcnt2 + (1 - oob[jl])
            return pcnt2

        pcnt = lax.fori_loop(0, ngroups, group, pcnt)
        nblk = pcnt // GSUB

        def fb(b, fl2):
            pltpu.sync_copy(pend.at[pl.ds(b * GSUB, GSUB)],
                            lists_hbm.at[pl.ds(listbase + fl2 * GSUB, GSUB)])
            return fl2 + 1

        fl = lax.fori_loop(0, nblk, fb, fl)
        rbase = nblk * GSUB
        for g in range(GSUB // 16):
            pend[pl.ds(16 * g, 16)] = pend[pl.ds(rbase + 16 * g, 16)]
        return pcnt - rbase, fl

    pcnt, fl = lax.fori_loop(0, NSCAN, scan_chunk,
                             (jnp.int32(0), jnp.int32(0)))

    dummyv = jnp.full((16,), DUMMY, jnp.int32)
    for g in range(GSUB // 16):
        pend[pl.ds(pcnt + 16 * g, 16)] = dummyv

    @pl.when(pcnt > 0)
    def _():
        pltpu.sync_copy(pend.at[pl.ds(0, GSUB)],
                        lists_hbm.at[pl.ds(listbase + fl * GSUB, GSUB)])

    flf = jnp.where(pcnt > 0, fl + 1, fl)
    cntbuf[pl.ds(0, 16)] = jnp.broadcast_to(flf * GSUB, (16,))
    pltpu.sync_copy(cntbuf.at[pl.ds(0, 16)], counts_hbm.at[pl.ds(t * 16, 16)])

    ones16 = jnp.ones((16,), jnp.float32)

    def initrow(r, cy):
        acc16[r, :] = ones16
        return cy

    lax.fori_loop(0, SLAB + 1, initrow, 0)

    def degblk(b, cy):
        pltpu.sync_copy(lists_hbm.at[pl.ds(listbase + b * GSUB, GSUB)],
                        lbuf.at[pl.ds(0, GSUB)])

        def deggrp(g, cy2):
            ld16 = lbuf[pl.ds(16 * g, 16)] & (SHIFT - 1)
            for jl in range(16):
                ld = ld16[jl]
                acc16[ld, :] = acc16[ld, :] + ones16
            return cy2

        return lax.fori_loop(0, GSUB // 16, deggrp, cy)

    lax.fori_loop(0, flf, degblk, 0)

    @pl.when(t < NT - 1)
    def _():
        pltpu.sync_copy(acc16.at[pl.ds(0, SLAB)], deg_hbm.at[pl.ds(lo, SLAB)])

    @pl.when(t == NT - 1)
    def _():
        pltpu.sync_copy(acc16.at[pl.ds(0, LAST)], deg_hbm.at[pl.ds(lo, LAST)])


_sc_prep = pl.kernel(
    _prep_body,
    out_type=(
        jax.ShapeDtypeStruct((NT * CAP,), jnp.int32),
        jax.ShapeDtypeStruct((NT * 16,), jnp.int32),
        jax.ShapeDtypeStruct((N, 16), jnp.float32),
    ),
    mesh=_MESH,
    scratch_types=[
        pltpu.VMEM((CHE,), jnp.int32),
        pltpu.VMEM((CHE,), jnp.int32),
        pltpu.VMEM((PCAP,), jnp.int32),
        pltpu.VMEM((GSUB,), jnp.int32),
        pltpu.VMEM((16,), jnp.int32),
        pltpu.VMEM((SLAB + 1, 16), jnp.float32),
    ],
)


def _agg_body(hp_hbm, lists_hbm, counts_hbm, out_hbm,
              lbig, lbufa, lbufb, srca, srcb, cntbuf, rowsa, rowsb, acc,
              sema, semb):
    c = lax.axis_index("c")
    s = lax.axis_index("s")
    t = c * NS + s
    lo = t * SLAB
    listbase = t * CAP

    @pl.when(t < NT - 1)
    def _():
        pltpu.sync_copy(hp_hbm.at[pl.ds(lo, SLAB)], acc.at[pl.ds(0, SLAB)])

    @pl.when(t == NT - 1)
    def _():
        pltpu.sync_copy(hp_hbm.at[pl.ds(lo, LAST)], acc.at[pl.ds(0, LAST)])

    pltpu.sync_copy(counts_hbm.at[pl.ds(t * 16, 16)], cntbuf.at[pl.ds(0, 16)])
    nblk = cntbuf[pl.ds(0, 16)][0] // GSUB

    def issue(bb, lbuf, srcv, rows, sem):
        base = jnp.bitwise_and(bb, LCB - 1) * GSUB
        for g in range(GSUB // 16):
            v = lbig[pl.ds(base + 16 * g, 16)]
            lbuf[pl.ds(16 * g, 16)] = v & (SHIFT - 1)
            srcv[pl.ds(16 * g, 16)] = jnp.right_shift(v, 9)
        pltpu.async_copy(hp_hbm.at[srcv], rows, sem)

    def accum(lbuf, rows):
        def grp(g, cy2):
            ld16 = lbuf[pl.ds(16 * g, 16)]
            for jl in range(16):
                ld = ld16[jl]
                for f in range(H // 16):
                    sl = pl.ds(16 * f, 16)
                    acc[ld, sl] = acc[ld, sl] + rows[16 * g + jl, sl]
            return cy2

        lax.fori_loop(0, GSUB // 16, grp, 0)

    @pl.when(nblk > 0)
    def _():
        pltpu.sync_copy(lists_hbm.at[pl.ds(listbase, LCHUNK)],
                        lbig.at[pl.ds(0, LCHUNK)])
        issue(0, lbufa, srca, rowsa, sema)

    def blk(b, cy):
        p = jnp.bitwise_and(b, 1)
        nxt = b + 1

        @pl.when((nxt < nblk) & (jnp.bitwise_and(nxt, LCB - 1) == 0))
        def _():
            pltpu.sync_copy(
                lists_hbm.at[pl.ds(listbase + (nxt // LCB) * LCHUNK, LCHUNK)],
                lbig.at[pl.ds(0, LCHUNK)])

        @pl.when((nxt < nblk) & (p == 0))
        def _():
            issue(nxt, lbufb, srcb, rowsb, semb)

        @pl.when((nxt < nblk) & (p == 1))
        def _():
            issue(nxt, lbufa, srca, rowsa, sema)

        @pl.when(p == 0)
        def _():
            pltpu.make_async_copy(hp_hbm.at[srca], rowsa, sema).wait()
            accum(lbufa, rowsa)

        @pl.when(p == 1)
        def _():
            pltpu.make_async_copy(hp_hbm.at[srcb], rowsb, semb).wait()
            accum(lbufb, rowsb)

        return cy

    lax.fori_loop(0, nblk, blk, 0)

    @pl.when(t < NT - 1)
    def _():
        pltpu.sync_copy(acc.at[pl.ds(0, SLAB)], out_hbm.at[pl.ds(lo, SLAB)])

    @pl.when(t == NT - 1)
    def _():
        pltpu.sync_copy(acc.at[pl.ds(0, LAST)], out_hbm.at[pl.ds(lo, LAST)])


_sc_agg = pl.kernel(
    _agg_body,
    out_type=jax.ShapeDtypeStruct((N, H), jnp.float32),
    mesh=_MESH,
    scratch_types=[
        pltpu.VMEM((LCHUNK,), jnp.int32),
        pltpu.VMEM((GSUB,), jnp.int32),
        pltpu.VMEM((GSUB,), jnp.int32),
        pltpu.VMEM((GSUB,), jnp.int32),
        pltpu.VMEM((GSUB,), jnp.int32),
        pltpu.VMEM((16,), jnp.int32),
        pltpu.VMEM((GSUB, H), jnp.float32),
        pltpu.VMEM((GSUB, H), jnp.float32),
        pltpu.VMEM((SLAB + 1, H), jnp.float32),
        pltpu.SemaphoreType.DMA,
        pltpu.SemaphoreType.DMA,
    ],
)

BN = 1000
GRID = N // BN


def _first_body(x_ref, w_ref, deg_ref, hp_ref, dinv_ref):
    dinv = lax.rsqrt(deg_ref[...])
    y = jnp.dot(x_ref[...], w_ref[...], preferred_element_type=jnp.float32,
                precision=_HIGH)
    hp_ref[...] = y * dinv
    dinv_ref[...] = dinv


_tc_first = pl.pallas_call(
    _first_body,
    grid=(GRID,),
    in_specs=[
        pl.BlockSpec((BN, F_IN), lambda i: (i, 0)),
        pl.BlockSpec((F_IN, H), lambda i: (0, 0)),
        pl.BlockSpec((BN, 1), lambda i: (i, 0)),
    ],
    out_specs=[
        pl.BlockSpec((BN, H), lambda i: (i, 0)),
        pl.BlockSpec((BN, 1), lambda i: (i, 0)),
    ],
    out_shape=[
        jax.ShapeDtypeStruct((N, H), jnp.float32),
        jax.ShapeDtypeStruct((N, 1), jnp.float32),
    ],
)


def _layer_body(s_ref, dinv_ref, b_ref, w_ref, o_ref):
    dinv = dinv_ref[...]
    v = dinv * s_ref[...] + b_ref[...]
    z = jnp.where(v >= 0, v, ALPHA * v)
    y = jnp.dot(z, w_ref[...], preferred_element_type=jnp.float32, precision=_HIGH)
    o_ref[...] = y * dinv


_tc_layer = pl.pallas_call(
    _layer_body,
    grid=(GRID,),
    in_specs=[
        pl.BlockSpec((BN, H), lambda i: (i, 0)),
        pl.BlockSpec((BN, 1), lambda i: (i, 0)),
        pl.BlockSpec((1, H), lambda i: (0, 0)),
        pl.BlockSpec((H, H), lambda i: (0, 0)),
    ],
    out_specs=pl.BlockSpec((BN, H), lambda i: (i, 0)),
    out_shape=jax.ShapeDtypeStruct((N, H), jnp.float32),
)


def _final_body(s_ref, dinv_ref, b_ref, bid_ref, wl1_ref, wl2_ref, bl_ref,
                o_ref, maxa, suma, cnta):
    i = pl.program_id(0)

    @pl.when(i == 0)
    def _():
        maxa[...] = jnp.full((G, H), -jnp.inf, jnp.float32)
        suma[...] = jnp.zeros((G, H), jnp.float32)
        cnta[...] = jnp.zeros((G, 1), jnp.float32)

    v = dinv_ref[...] * s_ref[...] + b_ref[...]
    h = jnp.where(v >= 0, v, ALPHA * v)
    bid = bid_ref[...]
    gids = lax.broadcasted_iota(jnp.int32, (BN, G), 1)
    mask = (bid == gids).astype(jnp.float32)
    suma[...] += lax.dot_general(mask, h, (((0,), (0,)), ((), ())),
                                 preferred_element_type=jnp.float32,
                                 precision=_HIGH)
    onesc = jnp.ones((BN, 1), jnp.float32)
    cnta[...] += lax.dot_general(mask, onesc, (((0,), (0,)), ((), ())),
                                 preferred_element_type=jnp.float32,
                                 precision=_HIGH)
    for g in range(G):
        m = jnp.max(jnp.where(bid == g, h, -jnp.inf), axis=0, keepdims=True)
        maxa[pl.ds(g, 1), :] = jnp.maximum(maxa[pl.ds(g, 1), :], m)

    @pl.when(i == GRID - 1)
    def _():
        maxp = maxa[...]
        maxp = jnp.where(jnp.isfinite(maxp), maxp, 0.0)
        meanp = suma[...] / jnp.maximum(cnta[...], 1.0)
        o_ref[...] = (
            jnp.dot(maxp, wl1_ref[...], preferred_element_type=jnp.float32,
                    precision=_HIGH)
            + jnp.dot(meanp, wl2_ref[...], preferred_element_type=jnp.float32,
                      precision=_HIGH)
            + bl_ref[...]
        )


_tc_final = pl.pallas_call(
    _final_body,
    grid=(GRID,),
    in_specs=[
        pl.BlockSpec((BN, H), lambda i: (i, 0)),
        pl.BlockSpec((BN, 1), lambda i: (i, 0)),
        pl.BlockSpec((1, H), lambda i: (0, 0)),
        pl.BlockSpec((BN, 1), lambda i: (i, 0)),
        pl.BlockSpec((H, C), lambda i: (0, 0)),
        pl.BlockSpec((H, C), lambda i: (0, 0)),
        pl.BlockSpec((1, C), lambda i: (0, 0)),
    ],
    out_specs=pl.BlockSpec((G, C), lambda i: (0, 0)),
    out_shape=jax.ShapeDtypeStruct((G, C), jnp.float32),
    scratch_shapes=[
        pltpu.VMEM((G, H), jnp.float32),
        pltpu.VMEM((G, H), jnp.float32),
        pltpu.VMEM((G, 1), jnp.float32),
    ],
)


def kernel(x, edge_index, batch, W0, b0, W1, b1, W2, b2, W3, b3, Wl, bl):
    src = edge_index[0]
    dst = edge_index[1]
    lists, counts, deg16 = _sc_prep(src, dst)
    deg = deg16[:, 0:1]
    hp, dinv = _tc_first(x, W0, deg)
    s = _sc_agg(hp, lists, counts)
    hp = _tc_layer(s, dinv, b0.reshape(1, H), W1)
    s = _sc_agg(hp, lists, counts)
    hp = _tc_layer(s, dinv, b1.reshape(1, H), W2)
    s = _sc_agg(hp, lists, counts)
    hp = _tc_layer(s, dinv, b2.reshape(1, H), W3)
    s = _sc_agg(hp, lists, counts)
    return _tc_final(s, dinv, b3.reshape(1, H), batch.reshape(N, 1),
                     Wl[:H], Wl[H:], bl.reshape(1, C))


# speedup vs baseline: 4.2841x; 1.0004x over previous
import jax
import jax.numpy as jnp
from jax import lax
from jax.experimental import pallas as pl
from jax.experimental.pallas import tpu as pltpu
from jax.experimental.pallas import tpu_sc as plsc

N = 10000
E = 320000
F_IN = 128
H = 256
G = 64
C = 10
ALPHA = 0.01

NC = 2
NS = 16
NT = NC * NS
SLAB = 320
LAST = N - (NT - 1) * SLAB
DUMMY = SLAB
SHIFT = 512
CHE = 2048
NSCAN = (E + CHE - 1) // CHE
REME = E - (NSCAN - 1) * CHE
GSUB = 64
PCAP = CHE + 2 * GSUB
LCHUNK = 1024
LCB = LCHUNK // GSUB
CAP = ((E + GSUB + LCHUNK - 1) // LCHUNK) * LCHUNK

_HIGH = lax.Precision.HIGHEST
_MESH = plsc.VectorSubcoreMesh(core_axis_name="c", subcore_axis_name="s")


def _prep_body(src_hbm, dst_hbm, lists_hbm, counts_hbm, deg_hbm,
               srcbuf, dstbuf, pend, lbuf, cntbuf, acc16):
    c = lax.axis_index("c")
    s = lax.axis_index("s")
    t = c * NS + s
    lo = t * SLAB
    listbase = t * CAP

    def scan_chunk(ci, carry):
        pcnt, fl = carry
        off = pl.multiple_of(ci * CHE, CHE)

        @pl.when(ci < NSCAN - 1)
        def _():
            pltpu.sync_copy(src_hbm.at[pl.ds(off, CHE)], srcbuf.at[pl.ds(0, CHE)])
            pltpu.sync_copy(dst_hbm.at[pl.ds(off, CHE)], dstbuf.at[pl.ds(0, CHE)])

        @pl.when(ci == NSCAN - 1)
        def _():
            pltpu.sync_copy(src_hbm.at[pl.ds(off, REME)], srcbuf.at[pl.ds(0, REME)])
            pltpu.sync_copy(dst_hbm.at[pl.ds(off, REME)], dstbuf.at[pl.ds(0, REME)])

        ngroups = jnp.where(ci < NSCAN - 1, CHE // 16, REME // 16)

        def group(j, pcnt2):
            d = dstbuf[pl.ds(16 * j, 16)]
            sv = srcbuf[pl.ds(16 * j, 16)]
            u = d - lo
            oob = lax.shift_right_logical(u | (SLAB - 1 - u), 31)
            comb = sv * SHIFT + u
            for jl in range(16):
                pend[pl.ds(pcnt2, 16)] = jnp.broadcast_to(comb[jl], (16,))
                pcnt2 = pcnt2 + (1 - oob[jl])
            return pcnt2

        pcnt = lax.fori_loop(0, ngroups, group, pcnt)
        nblk = pcnt // GSUB

        def fb(b, fl2):
            pltpu.sync_copy(pend.at[pl.ds(b * GSUB, GSUB)],
                            lists_hbm.at[pl.ds(listbase + fl2 * GSUB, GSUB)])
            return fl2 + 1

        fl = lax.fori_loop(0, nblk, fb, fl)
        rbase = nblk * GSUB
        for g in range(GSUB // 16):
            pend[pl.ds(16 * g, 16)] = pend[pl.ds(rbase + 16 * g, 16)]
        return pcnt - rbase, fl

    pcnt, fl = lax.fori_loop(0, NSCAN, scan_chunk,
                             (jnp.int32(0), jnp.int32(0)))

    dummyv = jnp.full((16,), DUMMY, jnp.int32)
    for g in range(GSUB // 16):
        pend[pl.ds(pcnt + 16 * g, 16)] = dummyv

    @pl.when(pcnt > 0)
    def _():
        pltpu.sync_copy(pend.at[pl.ds(0, GSUB)],
                        lists_hbm.at[pl.ds(listbase + fl * GSUB, GSUB)])

    flf = jnp.where(pcnt > 0, fl + 1, fl)
    cntbuf[pl.ds(0, 16)] = jnp.broadcast_to(flf * GSUB, (16,))
    pltpu.sync_copy(cntbuf.at[pl.ds(0, 16)], counts_hbm.at[pl.ds(t * 16, 16)])

    ones16 = jnp.ones((16,), jnp.float32)

    def initrow(r, cy):
        acc16[r, :] = ones16
        return cy

    lax.fori_loop(0, SLAB + 1, initrow, 0)

    def degblk(b, cy):
        pltpu.sync_copy(lists_hbm.at[pl.ds(listbase + b * GSUB, GSUB)],
                        lbuf.at[pl.ds(0, GSUB)])

        def deggrp(g, cy2):
            ld16 = lbuf[pl.ds(16 * g, 16)] & (SHIFT - 1)
            for jl in range(16):
                ld = ld16[jl]
                acc16[ld, :] = acc16[ld, :] + ones16
            return cy2

        return lax.fori_loop(0, GSUB // 16, deggrp, cy)

    lax.fori_loop(0, flf, degblk, 0)

    @pl.when(t < NT - 1)
    def _():
        pltpu.sync_copy(acc16.at[pl.ds(0, SLAB)], deg_hbm.at[pl.ds(lo, SLAB)])

    @pl.when(t == NT - 1)
    def _():
        pltpu.sync_copy(acc16.at[pl.ds(0, LAST)], deg_hbm.at[pl.ds(lo, LAST)])


_sc_prep = pl.kernel(
    _prep_body,
    out_type=(
        jax.ShapeDtypeStruct((NT * CAP,), jnp.int32),
        jax.ShapeDtypeStruct((NT * 16,), jnp.int32),
        jax.ShapeDtypeStruct((N, 16), jnp.float32),
    ),
    mesh=_MESH,
    scratch_types=[
        pltpu.VMEM((CHE,), jnp.int32),
        pltpu.VMEM((CHE,), jnp.int32),
        pltpu.VMEM((PCAP,), jnp.int32),
        pltpu.VMEM((GSUB,), jnp.int32),
        pltpu.VMEM((16,), jnp.int32),
        pltpu.VMEM((SLAB + 1, 16), jnp.float32),
    ],
)


def _agg_body(hp_hbm, lists_hbm, counts_hbm, out_hbm,
              lbig, lbufa, lbufb, srca, srcb, cntbuf, rowsa, rowsb, acc,
              sema, semb, semc, semd):
    c = lax.axis_index("c")
    s = lax.axis_index("s")
    t = c * NS + s
    lo = t * SLAB
    listbase = t * CAP

    @pl.when(t < NT - 1)
    def _():
        pltpu.sync_copy(hp_hbm.at[pl.ds(lo, SLAB)], acc.at[pl.ds(0, SLAB)])

    @pl.when(t == NT - 1)
    def _():
        pltpu.sync_copy(hp_hbm.at[pl.ds(lo, LAST)], acc.at[pl.ds(0, LAST)])

    pltpu.sync_copy(counts_hbm.at[pl.ds(t * 16, 16)], cntbuf.at[pl.ds(0, 16)])
    nblk = cntbuf[pl.ds(0, 16)][0] // GSUB

    HG = GSUB // 2

    def issue(bb, lbuf, srcv, rows, sem, sem2):
        base = jnp.bitwise_and(bb, LCB - 1) * GSUB
        for g in range(GSUB // 16):
            v = lbig[pl.ds(base + 16 * g, 16)]
            lbuf[pl.ds(16 * g, 16)] = v & (SHIFT - 1)
            srcv[pl.ds(16 * g, 16)] = jnp.right_shift(v, 9)
        pltpu.async_copy(hp_hbm.at[srcv.at[pl.ds(0, HG)]],
                         rows.at[pl.ds(0, HG)], sem)
        pltpu.async_copy(hp_hbm.at[srcv.at[pl.ds(HG, HG)]],
                         rows.at[pl.ds(HG, HG)], sem2)

    def wait_pair(srcv, rows, sem, sem2):
        pltpu.make_async_copy(hp_hbm.at[srcv.at[pl.ds(0, HG)]],
                              rows.at[pl.ds(0, HG)], sem).wait()
        pltpu.make_async_copy(hp_hbm.at[srcv.at[pl.ds(HG, HG)]],
                              rows.at[pl.ds(HG, HG)], sem2).wait()

    def accum(lbuf, rows):
        def grp(g, cy2):
            ld16 = lbuf[pl.ds(16 * g, 16)]
            for jl in range(16):
                ld = ld16[jl]
                for f in range(H // 16):
                    sl = pl.ds(16 * f, 16)
                    acc[ld, sl] = acc[ld, sl] + rows[16 * g + jl, sl]
            return cy2

        lax.fori_loop(0, GSUB // 16, grp, 0)

    @pl.when(nblk > 0)
    def _():
        pltpu.sync_copy(lists_hbm.at[pl.ds(listbase, LCHUNK)],
                        lbig.at[pl.ds(0, LCHUNK)])
        issue(0, lbufa, srca, rowsa, sema, semc)

    def blk(b, cy):
        p = jnp.bitwise_and(b, 1)
        nxt = b + 1

        @pl.when((nxt < nblk) & (jnp.bitwise_and(nxt, LCB - 1) == 0))
        def _():
            pltpu.sync_copy(
                lists_hbm.at[pl.ds(listbase + (nxt // LCB) * LCHUNK, LCHUNK)],
                lbig.at[pl.ds(0, LCHUNK)])

        @pl.when((nxt < nblk) & (p == 0))
        def _():
            issue(nxt, lbufb, srcb, rowsb, semb, semd)

        @pl.when((nxt < nblk) & (p == 1))
        def _():
            issue(nxt, lbufa, srca, rowsa, sema, semc)

        @pl.when(p == 0)
        def _():
            wait_pair(srca, rowsa, sema, semc)
            accum(lbufa, rowsa)

        @pl.when(p == 1)
        def _():
            wait_pair(srcb, rowsb, semb, semd)
            accum(lbufb, rowsb)

        return cy

    lax.fori_loop(0, nblk, blk, 0)

    @pl.when(t < NT - 1)
    def _():
        pltpu.sync_copy(acc.at[pl.ds(0, SLAB)], out_hbm.at[pl.ds(lo, SLAB)])

    @pl.when(t == NT - 1)
    def _():
        pltpu.sync_copy(acc.at[pl.ds(0, LAST)], out_hbm.at[pl.ds(lo, LAST)])


_sc_agg = pl.kernel(
    _agg_body,
    out_type=jax.ShapeDtypeStruct((N, H), jnp.float32),
    mesh=_MESH,
    scratch_types=[
        pltpu.VMEM((LCHUNK,), jnp.int32),
        pltpu.VMEM((GSUB,), jnp.int32),
        pltpu.VMEM((GSUB,), jnp.int32),
        pltpu.VMEM((GSUB,), jnp.int32),
        pltpu.VMEM((GSUB,), jnp.int32),
        pltpu.VMEM((16,), jnp.int32),
        pltpu.VMEM((GSUB, H), jnp.float32),
        pltpu.VMEM((GSUB, H), jnp.float32),
        pltpu.VMEM((SLAB + 1, H), jnp.float32),
        pltpu.SemaphoreType.DMA,
        pltpu.SemaphoreType.DMA,
        pltpu.SemaphoreType.DMA,
        pltpu.SemaphoreType.DMA,
    ],
)

BN = 1000
GRID = N // BN


def _first_body(x_ref, w_ref, deg_ref, hp_ref, dinv_ref):
    dinv = lax.rsqrt(deg_ref[...])
    y = jnp.dot(x_ref[...], w_ref[...], preferred_element_type=jnp.float32,
                precision=_HIGH)
    hp_ref[...] = y * dinv
    dinv_ref[...] = dinv


_tc_first = pl.pallas_call(
    _first_body,
    grid=(GRID,),
    in_specs=[
        pl.BlockSpec((BN, F_IN), lambda i: (i, 0)),
        pl.BlockSpec((F_IN, H), lambda i: (0, 0)),
        pl.BlockSpec((BN, 1), lambda i: (i, 0)),
    ],
    out_specs=[
        pl.BlockSpec((BN, H), lambda i: (i, 0)),
        pl.BlockSpec((BN, 1), lambda i: (i, 0)),
    ],
    out_shape=[
        jax.ShapeDtypeStruct((N, H), jnp.float32),
        jax.ShapeDtypeStruct((N, 1), jnp.float32),
    ],
)


def _layer_body(s_ref, dinv_ref, b_ref, w_ref, o_ref):
    dinv = dinv_ref[...]
    v = dinv * s_ref[...] + b_ref[...]
    z = jnp.where(v >= 0, v, ALPHA * v)
    y = jnp.dot(z, w_ref[...], preferred_element_type=jnp.float32, precision=_HIGH)
    o_ref[...] = y * dinv


_tc_layer = pl.pallas_call(
    _layer_body,
    grid=(GRID,),
    in_specs=[
        pl.BlockSpec((BN, H), lambda i: (i, 0)),
        pl.BlockSpec((BN, 1), lambda i: (i, 0)),
        pl.BlockSpec((1, H), lambda i: (0, 0)),
        pl.BlockSpec((H, H), lambda i: (0, 0)),
    ],
    out_specs=pl.BlockSpec((BN, H), lambda i: (i, 0)),
    out_shape=jax.ShapeDtypeStruct((N, H), jnp.float32),
)


def _final_body(s_ref, dinv_ref, b_ref, bid_ref, wl1_ref, wl2_ref, bl_ref,
                o_ref, maxa, suma, cnta):
    i = pl.program_id(0)

    @pl.when(i == 0)
    def _():
        maxa[...] = jnp.full((G, H), -jnp.inf, jnp.float32)
        suma[...] = jnp.zeros((G, H), jnp.float32)
        cnta[...] = jnp.zeros((G, 1), jnp.float32)

    v = dinv_ref[...] * s_ref[...] + b_ref[...]
    h = jnp.where(v >= 0, v, ALPHA * v)
    bid = bid_ref[...]
    gids = lax.broadcasted_iota(jnp.int32, (BN, G), 1)
    mask = (bid == gids).astype(jnp.float32)
    suma[...] += lax.dot_general(mask, h, (((0,), (0,)), ((), ())),
                                 preferred_element_type=jnp.float32,
                                 precision=_HIGH)
    onesc = jnp.ones((BN, 1), jnp.float32)
    cnta[...] += lax.dot_general(mask, onesc, (((0,), (0,)), ((), ())),
                                 preferred_element_type=jnp.float32,
                                 precision=_HIGH)
    for g in range(G):
        m = jnp.max(jnp.where(bid == g, h, -jnp.inf), axis=0, keepdims=True)
        maxa[pl.ds(g, 1), :] = jnp.maximum(maxa[pl.ds(g, 1), :], m)

    @pl.when(i == GRID - 1)
    def _():
        maxp = maxa[...]
        maxp = jnp.where(jnp.isfinite(maxp), maxp, 0.0)
        meanp = suma[...] / jnp.maximum(cnta[...], 1.0)
        o_ref[...] = (
            jnp.dot(maxp, wl1_ref[...], preferred_element_type=jnp.float32,
                    precision=_HIGH)
            + jnp.dot(meanp, wl2_ref[...], preferred_element_type=jnp.float32,
                      precision=_HIGH)
            + bl_ref[...]
        )


_tc_final = pl.pallas_call(
    _final_body,
    grid=(GRID,),
    in_specs=[
        pl.BlockSpec((BN, H), lambda i: (i, 0)),
        pl.BlockSpec((BN, 1), lambda i: (i, 0)),
        pl.BlockSpec((1, H), lambda i: (0, 0)),
        pl.BlockSpec((BN, 1), lambda i: (i, 0)),
        pl.BlockSpec((H, C), lambda i: (0, 0)),
        pl.BlockSpec((H, C), lambda i: (0, 0)),
        pl.BlockSpec((1, C), lambda i: (0, 0)),
    ],
    out_specs=pl.BlockSpec((G, C), lambda i: (0, 0)),
    out_shape=jax.ShapeDtypeStruct((G, C), jnp.float32),
    scratch_shapes=[
        pltpu.VMEM((G, H), jnp.float32),
        pltpu.VMEM((G, H), jnp.float32),
        pltpu.VMEM((G, 1), jnp.float32),
    ],
)


def kernel(x, edge_index, batch, W0, b0, W1, b1, W2, b2, W3, b3, Wl, bl):
    src = edge_index[0]
    dst = edge_index[1]
    lists, counts, deg16 = _sc_prep(src, dst)
    deg = deg16[:, 0:1]
    hp, dinv = _tc_first(x, W0, deg)
    s = _sc_agg(hp, lists, counts)
    hp = _tc_layer(s, dinv, b0.reshape(1, H), W1)
    s = _sc_agg(hp, lists, counts)
    hp = _tc_layer(s, dinv, b1.reshape(1, H), W2)
    s = _sc_agg(hp, lists, counts)
    hp = _tc_layer(s, dinv, b2.reshape(1, H), W3)
    s = _sc_agg(hp, lists, counts)
    return _tc_final(s, dinv, b3.reshape(1, H), batch.reshape(N, 1),
                     Wl[:H], Wl[H:], bl.reshape(1, C))


# speedup vs baseline: 4.8637x; 1.1353x over previous
import jax
import jax.numpy as jnp
import numpy as np
from jax import lax
from jax.experimental import pallas as pl
from jax.experimental.pallas import tpu as pltpu
from jax.experimental.pallas import tpu_sc as plsc

N = 10000
E = 320000
F_IN = 128
H = 256
G = 64
C = 10
ALPHA = 0.01

NC = 2
NS = 16
NT = NC * NS
SLAB = 320
LAST = N - (NT - 1) * SLAB
DUMMY = SLAB
SHIFT = 512
CHE = 2048
NSCAN = (E + CHE - 1) // CHE
REME = E - (NSCAN - 1) * CHE
GSUB = 64
PCAP = CHE + 2 * GSUB
LCHUNK = 1024
LCB = LCHUNK // GSUB
CAP = ((E + GSUB + LCHUNK - 1) // LCHUNK) * LCHUNK

_HIGH = lax.Precision.HIGHEST
_MESH = plsc.VectorSubcoreMesh(core_axis_name="c", subcore_axis_name="s")


def _prep_body(src_hbm, dst_hbm, lists_hbm, counts_hbm, deg_hbm,
               srcbuf, dstbuf, pend, lbuf, cntbuf, acc16):
    c = lax.axis_index("c")
    s = lax.axis_index("s")
    t = c * NS + s
    lo = t * SLAB
    listbase = t * CAP

    def scan_chunk(ci, carry):
        pcnt, fl = carry
        off = pl.multiple_of(ci * CHE, CHE)

        @pl.when(ci < NSCAN - 1)
        def _():
            pltpu.sync_copy(src_hbm.at[pl.ds(off, CHE)], srcbuf.at[pl.ds(0, CHE)])
            pltpu.sync_copy(dst_hbm.at[pl.ds(off, CHE)], dstbuf.at[pl.ds(0, CHE)])

        @pl.when(ci == NSCAN - 1)
        def _():
            pltpu.sync_copy(src_hbm.at[pl.ds(off, REME)], srcbuf.at[pl.ds(0, REME)])
            pltpu.sync_copy(dst_hbm.at[pl.ds(off, REME)], dstbuf.at[pl.ds(0, REME)])

        ngroups = jnp.where(ci < NSCAN - 1, CHE // 16, REME // 16)

        def group(j, pcnt2):
            d = dstbuf[pl.ds(16 * j, 16)]
            sv = srcbuf[pl.ds(16 * j, 16)]
            u = d - lo
            oob = lax.shift_right_logical(u | (SLAB - 1 - u), 31)
            comb = sv * SHIFT + u
            for jl in range(16):
                pend[pl.ds(pcnt2, 16)] = jnp.broadcast_to(comb[jl], (16,))
                pcnt2 = pcnt2 + (1 - oob[jl])
            return pcnt2

        pcnt = lax.fori_loop(0, ngroups, group, pcnt)
        nblk = pcnt // GSUB

        def fb(b, fl2):
            pltpu.sync_copy(pend.at[pl.ds(b * GSUB, GSUB)],
                            lists_hbm.at[pl.ds(listbase + fl2 * GSUB, GSUB)])
            return fl2 + 1

        fl = lax.fori_loop(0, nblk, fb, fl)
        rbase = nblk * GSUB
        for g in range(GSUB // 16):
            pend[pl.ds(16 * g, 16)] = pend[pl.ds(rbase + 16 * g, 16)]
        return pcnt - rbase, fl

    pcnt, fl = lax.fori_loop(0, NSCAN, scan_chunk,
                             (jnp.int32(0), jnp.int32(0)))

    dummyv = jnp.full((16,), DUMMY, jnp.int32)
    for g in range(GSUB // 16):
        pend[pl.ds(pcnt + 16 * g, 16)] = dummyv

    @pl.when(pcnt > 0)
    def _():
        pltpu.sync_copy(pend.at[pl.ds(0, GSUB)],
                        lists_hbm.at[pl.ds(listbase + fl * GSUB, GSUB)])

    flf = jnp.where(pcnt > 0, fl + 1, fl)
    cntbuf[pl.ds(0, 16)] = jnp.broadcast_to(flf * GSUB, (16,))
    pltpu.sync_copy(cntbuf.at[pl.ds(0, 16)], counts_hbm.at[pl.ds(t * 16, 16)])

    ones16 = jnp.ones((16,), jnp.float32)

    def initrow(r, cy):
        acc16[r, :] = ones16
        return cy

    lax.fori_loop(0, SLAB + 1, initrow, 0)

    def degblk(b, cy):
        pltpu.sync_copy(lists_hbm.at[pl.ds(listbase + b * GSUB, GSUB)],
                        lbuf.at[pl.ds(0, GSUB)])

        def deggrp(g, cy2):
            ld16 = lbuf[pl.ds(16 * g, 16)] & (SHIFT - 1)
            for jl in range(16):
                ld = ld16[jl]
                acc16[ld, :] = acc16[ld, :] + ones16
            return cy2

        return lax.fori_loop(0, GSUB // 16, deggrp, cy)

    lax.fori_loop(0, flf, degblk, 0)

    @pl.when(t < NT - 1)
    def _():
        pltpu.sync_copy(acc16.at[pl.ds(0, SLAB)], deg_hbm.at[pl.ds(lo, SLAB)])

    @pl.when(t == NT - 1)
    def _():
        pltpu.sync_copy(acc16.at[pl.ds(0, LAST)], deg_hbm.at[pl.ds(lo, LAST)])


_sc_prep = pl.kernel(
    _prep_body,
    out_type=(
        jax.ShapeDtypeStruct((NT * CAP,), jnp.int32),
        jax.ShapeDtypeStruct((NT * 16,), jnp.int32),
        jax.ShapeDtypeStruct((N, 16), jnp.float32),
    ),
    mesh=_MESH,
    scratch_types=[
        pltpu.VMEM((CHE,), jnp.int32),
        pltpu.VMEM((CHE,), jnp.int32),
        pltpu.VMEM((PCAP,), jnp.int32),
        pltpu.VMEM((GSUB,), jnp.int32),
        pltpu.VMEM((16,), jnp.int32),
        pltpu.VMEM((SLAB + 1, 16), jnp.float32),
    ],
)


def _agg_body(hp_hbm, lists_hbm, counts_hbm, out_hbm,
              lbig, lbufa, lbufb, srca, srcb, cntbuf, rowsa, rowsb, acc,
              sema, semb, semc, semd):
    c = lax.axis_index("c")
    s = lax.axis_index("s")
    t = c * NS + s
    lo = t * SLAB
    listbase = t * CAP

    himask0 = jnp.full((16,), -65536, jnp.int32)

    def initchunk(r0, nrows):
        pltpu.sync_copy(hp_hbm.at[pl.ds(lo + r0, nrows)],
                        rowsa.at[pl.ds(0, nrows)])

        def irow(r, cy):
            for f in range(H // 32):
                w = rowsa[r, pl.ds(16 * f, 16)]
                ev = lax.bitcast_convert_type(jnp.left_shift(w, 16), jnp.float32)
                od = lax.bitcast_convert_type(w & himask0, jnp.float32)
                acc[r0 + r, pl.ds(32 * f, 16)] = ev
                acc[r0 + r, pl.ds(32 * f + 16, 16)] = od
            return cy

        lax.fori_loop(0, nrows, irow, 0)

    @pl.when(t < NT - 1)
    def _():
        for k in range(SLAB // GSUB):
            initchunk(k * GSUB, GSUB)

    @pl.when(t == NT - 1)
    def _():
        initchunk(0, GSUB)
        initchunk(GSUB, LAST - GSUB)

    pltpu.sync_copy(counts_hbm.at[pl.ds(t * 16, 16)], cntbuf.at[pl.ds(0, 16)])
    nblk = cntbuf[pl.ds(0, 16)][0] // GSUB

    HG = GSUB // 2

    def issue(bb, lbuf, srcv, rows, sem, sem2):
        base = jnp.bitwise_and(bb, LCB - 1) * GSUB
        for g in range(GSUB // 16):
            v = lbig[pl.ds(base + 16 * g, 16)]
            lbuf[pl.ds(16 * g, 16)] = v & (SHIFT - 1)
            srcv[pl.ds(16 * g, 16)] = jnp.right_shift(v, 9)
        pltpu.async_copy(hp_hbm.at[srcv.at[pl.ds(0, HG)]],
                         rows.at[pl.ds(0, HG)], sem)
        pltpu.async_copy(hp_hbm.at[srcv.at[pl.ds(HG, HG)]],
                         rows.at[pl.ds(HG, HG)], sem2)

    def wait_pair(srcv, rows, sem, sem2):
        pltpu.make_async_copy(hp_hbm.at[srcv.at[pl.ds(0, HG)]],
                              rows.at[pl.ds(0, HG)], sem).wait()
        pltpu.make_async_copy(hp_hbm.at[srcv.at[pl.ds(HG, HG)]],
                              rows.at[pl.ds(HG, HG)], sem2).wait()

    himask = jnp.full((16,), -65536, jnp.int32)

    def accum(lbuf, rows):
        def grp(g, cy2):
            ld16 = lbuf[pl.ds(16 * g, 16)]
            for jl in range(16):
                ld = ld16[jl]
                for f in range(H // 32):
                    w = rows[16 * g + jl, pl.ds(16 * f, 16)]
                    ev = lax.bitcast_convert_type(jnp.left_shift(w, 16), jnp.float32)
                    od = lax.bitcast_convert_type(w & himask, jnp.float32)
                    sle = pl.ds(32 * f, 16)
                    slo = pl.ds(32 * f + 16, 16)
                    acc[ld, sle] = acc[ld, sle] + ev
                    acc[ld, slo] = acc[ld, slo] + od
            return cy2

        lax.fori_loop(0, GSUB // 16, grp, 0)

    @pl.when(nblk > 0)
    def _():
        pltpu.sync_copy(lists_hbm.at[pl.ds(listbase, LCHUNK)],
                        lbig.at[pl.ds(0, LCHUNK)])
        issue(0, lbufa, srca, rowsa, sema, semc)

    def blk(b, cy):
        p = jnp.bitwise_and(b, 1)
        nxt = b + 1

        @pl.when((nxt < nblk) & (jnp.bitwise_and(nxt, LCB - 1) == 0))
        def _():
            pltpu.sync_copy(
                lists_hbm.at[pl.ds(listbase + (nxt // LCB) * LCHUNK, LCHUNK)],
                lbig.at[pl.ds(0, LCHUNK)])

        @pl.when((nxt < nblk) & (p == 0))
        def _():
            issue(nxt, lbufb, srcb, rowsb, semb, semd)

        @pl.when((nxt < nblk) & (p == 1))
        def _():
            issue(nxt, lbufa, srca, rowsa, sema, semc)

        @pl.when(p == 0)
        def _():
            wait_pair(srca, rowsa, sema, semc)
            accum(lbufa, rowsa)

        @pl.when(p == 1)
        def _():
            wait_pair(srcb, rowsb, semb, semd)
            accum(lbufb, rowsb)

        return cy

    lax.fori_loop(0, nblk, blk, 0)

    @pl.when(t < NT - 1)
    def _():
        pltpu.sync_copy(acc.at[pl.ds(0, SLAB)], out_hbm.at[pl.ds(lo, SLAB)])

    @pl.when(t == NT - 1)
    def _():
        pltpu.sync_copy(acc.at[pl.ds(0, LAST)], out_hbm.at[pl.ds(lo, LAST)])


_sc_agg = pl.kernel(
    _agg_body,
    out_type=jax.ShapeDtypeStruct((N, H), jnp.float32),
    mesh=_MESH,
    scratch_types=[
        pltpu.VMEM((LCHUNK,), jnp.int32),
        pltpu.VMEM((GSUB,), jnp.int32),
        pltpu.VMEM((GSUB,), jnp.int32),
        pltpu.VMEM((GSUB,), jnp.int32),
        pltpu.VMEM((GSUB,), jnp.int32),
        pltpu.VMEM((16,), jnp.int32),
        pltpu.VMEM((GSUB, H // 2), jnp.int32),
        pltpu.VMEM((GSUB, H // 2), jnp.int32),
        pltpu.VMEM((SLAB + 1, H), jnp.float32),
        pltpu.SemaphoreType.DMA,
        pltpu.SemaphoreType.DMA,
        pltpu.SemaphoreType.DMA,
        pltpu.SemaphoreType.DMA,
    ],
)

BN = 1000
GRID = N // BN


def _first_body(x_ref, w_ref, deg_ref, hp_ref, dinv_ref):
    dinv = lax.rsqrt(deg_ref[...])
    y = jnp.dot(x_ref[...], w_ref[...], preferred_element_type=jnp.float32,
                precision=_HIGH)
    hp_ref[...] = (y * dinv).astype(jnp.bfloat16)
    dinv_ref[...] = dinv


_tc_first = pl.pallas_call(
    _first_body,
    grid=(GRID,),
    in_specs=[
        pl.BlockSpec((BN, F_IN), lambda i: (i, 0)),
        pl.BlockSpec((F_IN, H), lambda i: (0, 0)),
        pl.BlockSpec((BN, 1), lambda i: (i, 0)),
    ],
    out_specs=[
        pl.BlockSpec((BN, H), lambda i: (i, 0)),
        pl.BlockSpec((BN, 1), lambda i: (i, 0)),
    ],
    out_shape=[
        jax.ShapeDtypeStruct((N, H), jnp.bfloat16),
        jax.ShapeDtypeStruct((N, 1), jnp.float32),
    ],
)


def _layer_body(s_ref, dinv_ref, b_ref, w_ref, o_ref):
    dinv = dinv_ref[...]
    v = dinv * s_ref[...] + b_ref[...]
    z = jnp.where(v >= 0, v, ALPHA * v)
    y = jnp.dot(z, w_ref[...], preferred_element_type=jnp.float32, precision=_HIGH)
    o_ref[...] = (y * dinv).astype(jnp.bfloat16)


_tc_layer = pl.pallas_call(
    _layer_body,
    grid=(GRID,),
    in_specs=[
        pl.BlockSpec((BN, H), lambda i: (i, 0)),
        pl.BlockSpec((BN, 1), lambda i: (i, 0)),
        pl.BlockSpec((1, H), lambda i: (0, 0)),
        pl.BlockSpec((H, H), lambda i: (0, 0)),
    ],
    out_specs=pl.BlockSpec((BN, H), lambda i: (i, 0)),
    out_shape=jax.ShapeDtypeStruct((N, H), jnp.bfloat16),
)


def _final_body(s_ref, dinv_ref, b_ref, bid_ref, wl1_ref, wl2_ref,
                bl_ref, o_ref, maxa, suma, cnta):
    i = pl.program_id(0)

    @pl.when(i == 0)
    def _():
        maxa[...] = jnp.full((G, H), -jnp.inf, jnp.float32)
        suma[...] = jnp.zeros((G, H), jnp.float32)
        cnta[...] = jnp.zeros((G, 1), jnp.float32)

    v = dinv_ref[...] * s_ref[...] + b_ref[...]
    h = jnp.where(v >= 0, v, ALPHA * v)
    bid = bid_ref[...]
    gids = lax.broadcasted_iota(jnp.int32, (BN, G), 1)
    mask = (bid == gids).astype(jnp.float32)
    suma[...] += lax.dot_general(mask, h, (((0,), (0,)), ((), ())),
                                 preferred_element_type=jnp.float32,
                                 precision=_HIGH)
    onesc = jnp.ones((BN, 1), jnp.float32)
    cnta[...] += lax.dot_general(mask, onesc, (((0,), (0,)), ((), ())),
                                 preferred_element_type=jnp.float32,
                                 precision=_HIGH)
    for g in range(G):
        m = jnp.max(jnp.where(bid == g, h, -jnp.inf), axis=0, keepdims=True)
        maxa[pl.ds(g, 1), :] = jnp.maximum(maxa[pl.ds(g, 1), :], m)

    @pl.when(i == GRID - 1)
    def _():
        maxp = maxa[...]
        maxp = jnp.where(jnp.isfinite(maxp), maxp, 0.0)
        meanp = suma[...] / jnp.maximum(cnta[...], 1.0)
        o_ref[...] = (
            jnp.dot(maxp, wl1_ref[...], preferred_element_type=jnp.float32,
                    precision=_HIGH)
            + jnp.dot(meanp, wl2_ref[...], preferred_element_type=jnp.float32,
                      precision=_HIGH)
            + bl_ref[...]
        )


_tc_final = pl.pallas_call(
    _final_body,
    grid=(GRID,),
    in_specs=[
        pl.BlockSpec((BN, H), lambda i: (i, 0)),
        pl.BlockSpec((BN, 1), lambda i: (i, 0)),
        pl.BlockSpec((1, H), lambda i: (0, 0)),
        pl.BlockSpec((BN, 1), lambda i: (i, 0)),
        pl.BlockSpec((H, C), lambda i: (0, 0)),
        pl.BlockSpec((H, C), lambda i: (0, 0)),
        pl.BlockSpec((1, C), lambda i: (0, 0)),
    ],
    out_specs=pl.BlockSpec((G, C), lambda i: (0, 0)),
    out_shape=jax.ShapeDtypeStruct((G, C), jnp.float32),
    scratch_shapes=[
        pltpu.VMEM((G, H), jnp.float32),
        pltpu.VMEM((G, H), jnp.float32),
        pltpu.VMEM((G, 1), jnp.float32),
    ],
)


_PI = np.concatenate([
    np.concatenate([32 * f + np.arange(0, 32, 2), 32 * f + np.arange(1, 32, 2)])
    for f in range(H // 32)
]).astype(np.int32)


def _packed(hp):
    return jax.lax.bitcast_convert_type(hp.reshape(N, H // 2, 2), jnp.int32)


def kernel(x, edge_index, batch, W0, b0, W1, b1, W2, b2, W3, b3, Wl, bl):
    src = edge_index[0]
    dst = edge_index[1]
    lists, counts, deg16 = _sc_prep(src, dst)
    deg = deg16[:, 0:1]
    hp, dinv = _tc_first(x, W0, deg)
    s = _sc_agg(_packed(hp), lists, counts)
    hp = _tc_layer(s, dinv, b0[_PI].reshape(1, H), W1[_PI])
    s = _sc_agg(_packed(hp), lists, counts)
    hp = _tc_layer(s, dinv, b1[_PI].reshape(1, H), W2[_PI])
    s = _sc_agg(_packed(hp), lists, counts)
    hp = _tc_layer(s, dinv, b2[_PI].reshape(1, H), W3[_PI])
    s = _sc_agg(_packed(hp), lists, counts)
    return _tc_final(s, dinv, b3[_PI].reshape(1, H), batch.reshape(N, 1),
                     Wl[:H][_PI], Wl[H:][_PI], bl.reshape(1, C))
